# baseline (speedup 1.0000x reference)
import jax
import jax.numpy as jnp
from jax import lax
from jax.experimental import pallas as pl
from jax.experimental.pallas import tpu as pltpu

_B, _D, _H, _W = 8, 256, 32, 32
_ROWS_PER_BLOCK = 128
_BLOCKS_PER_PLANE = (2 * _D) // _ROWS_PER_BLOCK


def _pos_embed_kernel(row_ref, col_ref, out_ref, plane_ref):
    i = pl.program_id(0)

    @pl.when(i == 0)
    def _compute_plane():
        col = col_ref[:_W, :]
        row = row_ref[:_H, :]
        k = lax.broadcasted_iota(jnp.int32, (_W, _H * _W), 0)
        hw = lax.broadcasted_iota(jnp.int32, (_W, _H * _W), 1)
        sel_w = (hw % _W == k).astype(jnp.float32)
        sel_h = (hw // _W == k).astype(jnp.float32)
        dn = (((0,), (0,)), ((), ()))
        plane_ref[:_D, :] = lax.dot_general(
            col, sel_w, dn, preferred_element_type=jnp.float32)
        plane_ref[_D:, :] = lax.dot_general(
            row, sel_h, dn, preferred_element_type=jnp.float32)

    slot = i % _BLOCKS_PER_PLANE
    out_ref[...] = plane_ref[pl.ds(slot * _ROWS_PER_BLOCK, _ROWS_PER_BLOCK), :]


def kernel(pixel_values, row_embeddings, column_embeddings):
    B = pixel_values.shape[0]
    H = pixel_values.shape[-2]
    W = pixel_values.shape[-1]
    D = row_embeddings.shape[-1]
    n_blocks = (B * 2 * D) // _ROWS_PER_BLOCK
    out = pl.pallas_call(
        _pos_embed_kernel,
        grid=(n_blocks,),
        in_specs=[
            pl.BlockSpec(row_embeddings.shape, lambda i: (0, 0)),
            pl.BlockSpec(column_embeddings.shape, lambda i: (0, 0)),
        ],
        out_specs=pl.BlockSpec((_ROWS_PER_BLOCK, H * W), lambda i: (i, 0)),
        out_shape=jax.ShapeDtypeStruct((B * 2 * D, H * W), jnp.float32),
        scratch_shapes=[pltpu.VMEM((2 * D, H * W), jnp.float32)],
    )(row_embeddings, column_embeddings)
    return out.reshape(B, 2 * D, H, W)


# speedup vs baseline: 3.1913x; 3.1913x over previous
import jax
import jax.numpy as jnp
from jax import lax
from jax.experimental import pallas as pl
from jax.experimental.pallas import tpu as pltpu

_B, _D, _H, _W = 8, 256, 32, 32
_ROWS_PER_BLOCK = 128
_BLOCKS_PER_PLANE = (2 * _D) // _ROWS_PER_BLOCK


def _pos_embed_kernel(row_ref, col_ref, out_ref, p0_ref, p1_ref, p2_ref,
                      p3_ref, sem):
    col = col_ref[:_W, :]
    row = row_ref[:_H, :]
    k = lax.broadcasted_iota(jnp.int32, (_W, _H * _W), 0)
    hw = lax.broadcasted_iota(jnp.int32, (_W, _H * _W), 1)
    sel_w = (hw % _W == k).astype(jnp.float32)
    sel_h = (hw // _W == k).astype(jnp.float32)
    dn = (((0,), (0,)), ((), ()))
    x_part = lax.dot_general(col, sel_w, dn, preferred_element_type=jnp.float32)
    y_part = lax.dot_general(row, sel_h, dn, preferred_element_type=jnp.float32)
    planes = [p0_ref, p1_ref, p2_ref, p3_ref]
    for p in planes:
        p[:_D, :] = x_part
        p[_D:, :] = y_part
    copies = [
        pltpu.make_async_copy(planes[b % 4], out_ref.at[b], sem)
        for b in range(_B)
    ]
    for c in copies:
        c.start()
    for c in copies:
        c.wait()


def kernel(pixel_values, row_embeddings, column_embeddings):
    B = pixel_values.shape[0]
    H = pixel_values.shape[-2]
    W = pixel_values.shape[-1]
    D = row_embeddings.shape[-1]
    out = pl.pallas_call(
        _pos_embed_kernel,
        in_specs=[
            pl.BlockSpec(memory_space=pltpu.VMEM),
            pl.BlockSpec(memory_space=pltpu.VMEM),
        ],
        out_specs=pl.BlockSpec(memory_space=pl.ANY),
        out_shape=jax.ShapeDtypeStruct((B, 2 * D, H * W), jnp.float32),
        scratch_shapes=[
            pltpu.VMEM((2 * D, H * W), jnp.float32),
            pltpu.VMEM((2 * D, H * W), jnp.float32),
            pltpu.VMEM((2 * D, H * W), jnp.float32),
            pltpu.VMEM((2 * D, H * W), jnp.float32),
            pltpu.SemaphoreType.DMA,
        ],
    )(row_embeddings, column_embeddings)
    return out.reshape(B, 2 * D, H, W)
